# Initial kernel scaffold; baseline (speedup 1.0000x reference)
import jax
import jax.numpy as jnp
from jax.experimental import pallas as pl


def kernel(description_embedding, authors, publishers, categories, authors_table, publishers_table, categories_table, W1, b1, W2, b2):
    raise NotImplementedError("write your pallas kernel here")


# speedup vs baseline: 6.9436x; 6.9436x over previous
import functools

import jax
import jax.numpy as jnp
from jax import lax
from jax.experimental import pallas as pl
from jax.experimental.pallas import tpu as pltpu
from jax.experimental.pallas import tpu_sc as plsc

D = 128
NC, NS = 2, 16
NW = NC * NS
IDXW = 128


def _sc_gather(atab, ctab, ptab, aidx2d, cidx2d, pidx2d, dstidx, zeros, B, L):
    BPW = B // NW
    ROWS = BPW * L // IDXW
    PROWS = BPW // IDXW
    mesh = plsc.VectorSubcoreMesh(core_axis_name="c", subcore_axis_name="s")

    @functools.partial(
        pl.kernel,
        mesh=mesh,
        out_type=(
            jax.ShapeDtypeStruct((B, D), jnp.float32),
            jax.ShapeDtypeStruct((B, D), jnp.float32),
            jax.ShapeDtypeStruct((B, D), jnp.float32),
        ),
        scratch_types=[
            pltpu.VMEM((ROWS, IDXW), jnp.int32),
            pltpu.VMEM((ROWS, IDXW), jnp.int32),
            pltpu.VMEM((IDXW, D), jnp.float32),
            pltpu.VMEM_SHARED((NS * BPW, D), jnp.float32),
            pltpu.SemaphoreType.DMA,
            pltpu.SemaphoreType.DMA,
        ],
    )
    def k(atab_h, ctab_h, ptab_h, aidx_h, cidx_h, pidx_h, dst_h, z_h,
          a_out, p_out, c_out, idx_v, dst_v, buf, acc, sem, sem2):
        cix = lax.axis_index("c")
        six = lax.axis_index("s")
        wid = six * NC + cix
        base = wid * BPW
        pltpu.sync_copy(dst_h.at[six], dst_v)
        for tab_h, idx_h, out in ((atab_h, aidx_h, a_out), (ctab_h, cidx_h, c_out)):
            pltpu.sync_copy(z_h, acc.at[pl.ds(six * BPW, BPW)])
            pltpu.sync_copy(idx_h.at[pl.ds(wid * ROWS, ROWS)], idx_v)
            for j in range(ROWS):
                pltpu.async_copy(tab_h.at[idx_v.at[j]], buf, sem).wait()
                pltpu.async_copy(buf, acc.at[dst_v.at[j]], sem2, add=True).wait()
            pltpu.sync_copy(acc.at[pl.ds(six * BPW, BPW)], out.at[pl.ds(base, BPW)])
        pltpu.sync_copy(pidx_h.at[pl.ds(wid * PROWS, PROWS)],
                        idx_v.at[pl.ds(0, PROWS)])
        for j in range(PROWS):
            pltpu.async_copy(ptab_h.at[idx_v.at[j]], buf, sem).wait()
            pltpu.sync_copy(buf, p_out.at[pl.ds(base + j * IDXW, IDXW)])

    return k(atab, ctab, ptab, aidx2d, cidx2d, pidx2d, dstidx, zeros)


def _mlp(desc, asum, prow, csum, aidx, cidx, W1, b1, W2, b2):
    B, DD = desc.shape
    BM = 512
    L = aidx.shape[1]
    DIN, L1 = W1.shape
    L2 = W2.shape[1]

    def body(desc_ref, asum_ref, p_ref, csum_ref, aidx_ref, cidx_ref,
             W1_ref, b1_ref, W2_ref, b2_ref, out_ref):
        acnt = jnp.maximum(jnp.sum((aidx_ref[...] != 0).astype(jnp.float32),
                                   axis=1, keepdims=True), 1.0)
        ccnt = jnp.maximum(jnp.sum((cidx_ref[...] != 0).astype(jnp.float32),
                                   axis=1, keepdims=True), 1.0)
        a = asum_ref[...] / acnt
        c = csum_ref[...] / ccnt
        x = jnp.concatenate([desc_ref[...], a, p_ref[...], c], axis=1)
        h = jnp.dot(x, W1_ref[...], preferred_element_type=jnp.float32) + b1_ref[...]
        h = jnp.maximum(h, 0.0)
        out_ref[...] = jnp.dot(h, W2_ref[...],
                               preferred_element_type=jnp.float32) + b2_ref[...]

    return pl.pallas_call(
        body,
        grid=(B // BM,),
        in_specs=[
            pl.BlockSpec((BM, DD), lambda i: (i, 0)),
            pl.BlockSpec((BM, D), lambda i: (i, 0)),
            pl.BlockSpec((BM, D), lambda i: (i, 0)),
            pl.BlockSpec((BM, D), lambda i: (i, 0)),
            pl.BlockSpec((BM, L), lambda i: (i, 0)),
            pl.BlockSpec((BM, L), lambda i: (i, 0)),
            pl.BlockSpec((DIN, L1), lambda i: (0, 0)),
            pl.BlockSpec((1, L1), lambda i: (0, 0)),
            pl.BlockSpec((L1, L2), lambda i: (0, 0)),
            pl.BlockSpec((1, L2), lambda i: (0, 0)),
        ],
        out_specs=pl.BlockSpec((BM, L2), lambda i: (i, 0)),
        out_shape=jax.ShapeDtypeStruct((B, L2), jnp.float32),
    )(desc, asum, prow, csum, aidx, cidx, W1, b1.reshape(1, -1), W2,
      b2.reshape(1, -1))


def kernel(description_embedding, authors, publishers, categories,
           authors_table, publishers_table, categories_table,
           W1, b1, W2, b2):
    B, L = authors.shape
    aidx32 = authors.astype(jnp.int32)
    cidx32 = categories.astype(jnp.int32)
    aidx2d = aidx32.reshape(-1, IDXW)
    cidx2d = cidx32.reshape(-1, IDXW)
    pidx2d = publishers.astype(jnp.int32).reshape(-1, IDXW)
    BPW = B // NW
    ROWS = BPW * L // IDXW
    pat = (jnp.arange(BPW * L, dtype=jnp.int32) // L).reshape(ROWS, IDXW)
    dstidx = pat[None, :, :] + (jnp.arange(NS, dtype=jnp.int32) * BPW)[:, None, None]
    zeros = jnp.zeros((BPW, D), jnp.float32)
    asum, prow, csum = _sc_gather(authors_table, categories_table,
                                  publishers_table, aidx2d, cidx2d, pidx2d,
                                  dstidx, zeros, B, L)
    return _mlp(description_embedding, asum, prow, csum, aidx32, cidx32,
                W1, b1, W2, b2)


# speedup vs baseline: 9.0217x; 1.2993x over previous
import functools

import jax
import jax.numpy as jnp
from jax import lax
from jax.experimental import pallas as pl
from jax.experimental.pallas import tpu as pltpu
from jax.experimental.pallas import tpu_sc as plsc

D = 128
NC, NS = 2, 16
NW = NC * NS
IDXW = 128


def _sc_gather(atab, ctab, ptab, aidx2d, cidx2d, pidx2d, dstidx, zeros, B, L):
    BPW = B // NW
    ROWS = BPW * L // IDXW
    PROWS = BPW // IDXW
    mesh = plsc.VectorSubcoreMesh(core_axis_name="c", subcore_axis_name="s")

    @functools.partial(
        pl.kernel,
        mesh=mesh,
        out_type=(
            jax.ShapeDtypeStruct((B, D), jnp.float32),
            jax.ShapeDtypeStruct((B, D), jnp.float32),
            jax.ShapeDtypeStruct((B, D), jnp.float32),
        ),
        scratch_types=[
            pltpu.VMEM((ROWS, IDXW), jnp.int32),
            pltpu.VMEM((ROWS, IDXW), jnp.int32),
            pltpu.VMEM((ROWS, IDXW), jnp.int32),
            pltpu.VMEM((PROWS, IDXW), jnp.int32),
            pltpu.VMEM((IDXW, D), jnp.float32),
            pltpu.VMEM((IDXW, D), jnp.float32),
            pltpu.VMEM_SHARED((NS * BPW, D), jnp.float32),
            pltpu.SemaphoreType.DMA,
            pltpu.SemaphoreType.DMA,
            pltpu.SemaphoreType.DMA,
            pltpu.SemaphoreType.DMA,
            pltpu.SemaphoreType.DMA,
        ],
    )
    def k(atab_h, ctab_h, ptab_h, aidx_h, cidx_h, pidx_h, dst_h, z_h,
          a_out, p_out, c_out, idx_va, idx_vc, dst_v, idx_vp, buf0, buf1,
          acc, g0, g1, s0, s1, msem):
        cix = lax.axis_index("c")
        six = lax.axis_index("s")
        wid = six * NC + cix
        base = wid * BPW
        bufs = (buf0, buf1)
        gsem = (g0, g1)
        ssem = (s0, s1)
        accme = acc.at[pl.ds(six * BPW, BPW)]

        pre = [
            pltpu.async_copy(dst_h.at[six], dst_v, msem),
            pltpu.async_copy(aidx_h.at[pl.ds(wid * ROWS, ROWS)], idx_va, msem),
            pltpu.async_copy(cidx_h.at[pl.ds(wid * ROWS, ROWS)], idx_vc, msem),
            pltpu.async_copy(pidx_h.at[pl.ds(wid * PROWS, PROWS)], idx_vp, msem),
            pltpu.async_copy(z_h, accme, msem),
        ]
        for cp in pre:
            cp.wait()

        def run_pipe(n, start_gather, start_drain):
            g = [None] * n
            s = [None] * n
            g[0] = start_gather(0, bufs[0], gsem[0])
            for j in range(n):
                bi = j % 2
                if j + 1 < n:
                    if j >= 1:
                        s[j - 1].wait()
                    g[j + 1] = start_gather(j + 1, bufs[1 - bi], gsem[1 - bi])
                g[j].wait()
                s[j] = start_drain(j, bufs[bi], ssem[bi])
            if n >= 2:
                s[n - 2].wait()
            s[n - 1].wait()

        for tab_h, idx_v, out in ((atab_h, idx_va, a_out), (ctab_h, idx_vc, c_out)):
            run_pipe(
                ROWS,
                lambda j, b, sm, t=tab_h, iv=idx_v: pltpu.async_copy(
                    t.at[iv.at[j]], b, sm),
                lambda j, b, sm: pltpu.async_copy(
                    b, acc.at[dst_v.at[j]], sm, add=True),
            )
            pltpu.sync_copy(accme, out.at[pl.ds(base, BPW)])
            if tab_h is atab_h:
                pltpu.sync_copy(z_h, accme)
        run_pipe(
            PROWS,
            lambda j, b, sm: pltpu.async_copy(ptab_h.at[idx_vp.at[j]], b, sm),
            lambda j, b, sm: pltpu.async_copy(
                b, p_out.at[pl.ds(base + j * IDXW, IDXW)], sm),
        )

    return k(atab, ctab, ptab, aidx2d, cidx2d, pidx2d, dstidx, zeros)


def _mlp(desc, asum, prow, csum, aidx, cidx, W1, b1, W2, b2):
    B, DD = desc.shape
    BM = 512
    L = aidx.shape[1]
    DIN, L1 = W1.shape
    L2 = W2.shape[1]

    def body(desc_ref, asum_ref, p_ref, csum_ref, aidx_ref, cidx_ref,
             W1_ref, b1_ref, W2_ref, b2_ref, out_ref):
        acnt = jnp.maximum(jnp.sum((aidx_ref[...] != 0).astype(jnp.float32),
                                   axis=1, keepdims=True), 1.0)
        ccnt = jnp.maximum(jnp.sum((cidx_ref[...] != 0).astype(jnp.float32),
                                   axis=1, keepdims=True), 1.0)
        a = asum_ref[...] / acnt
        c = csum_ref[...] / ccnt
        x = jnp.concatenate([desc_ref[...], a, p_ref[...], c], axis=1)
        h = jnp.dot(x, W1_ref[...], preferred_element_type=jnp.float32) + b1_ref[...]
        h = jnp.maximum(h, 0.0)
        out_ref[...] = jnp.dot(h, W2_ref[...],
                               preferred_element_type=jnp.float32) + b2_ref[...]

    return pl.pallas_call(
        body,
        grid=(B // BM,),
        in_specs=[
            pl.BlockSpec((BM, DD), lambda i: (i, 0)),
            pl.BlockSpec((BM, D), lambda i: (i, 0)),
            pl.BlockSpec((BM, D), lambda i: (i, 0)),
            pl.BlockSpec((BM, D), lambda i: (i, 0)),
            pl.BlockSpec((BM, L), lambda i: (i, 0)),
            pl.BlockSpec((BM, L), lambda i: (i, 0)),
            pl.BlockSpec((DIN, L1), lambda i: (0, 0)),
            pl.BlockSpec((1, L1), lambda i: (0, 0)),
            pl.BlockSpec((L1, L2), lambda i: (0, 0)),
            pl.BlockSpec((1, L2), lambda i: (0, 0)),
        ],
        out_specs=pl.BlockSpec((BM, L2), lambda i: (i, 0)),
        out_shape=jax.ShapeDtypeStruct((B, L2), jnp.float32),
    )(desc, asum, prow, csum, aidx, cidx, W1, b1.reshape(1, -1), W2,
      b2.reshape(1, -1))


def kernel(description_embedding, authors, publishers, categories,
           authors_table, publishers_table, categories_table,
           W1, b1, W2, b2):
    B, L = authors.shape
    aidx32 = authors.astype(jnp.int32)
    cidx32 = categories.astype(jnp.int32)
    aidx2d = aidx32.reshape(-1, IDXW)
    cidx2d = cidx32.reshape(-1, IDXW)
    pidx2d = publishers.astype(jnp.int32).reshape(-1, IDXW)
    BPW = B // NW
    ROWS = BPW * L // IDXW
    pat = (jnp.arange(BPW * L, dtype=jnp.int32) // L).reshape(ROWS, IDXW)
    dstidx = pat[None, :, :] + (jnp.arange(NS, dtype=jnp.int32) * BPW)[:, None, None]
    zeros = jnp.zeros((BPW, D), jnp.float32)
    asum, prow, csum = _sc_gather(authors_table, categories_table,
                                  publishers_table, aidx2d, cidx2d, pidx2d,
                                  dstidx, zeros, B, L)
    return _mlp(description_embedding, asum, prow, csum, aidx32, cidx32,
                W1, b1, W2, b2)


# speedup vs baseline: 9.3094x; 1.0319x over previous
import functools

import jax
import jax.numpy as jnp
from jax import lax
from jax.experimental import pallas as pl
from jax.experimental.pallas import tpu as pltpu
from jax.experimental.pallas import tpu_sc as plsc

D = 128
NC, NS = 2, 16
NW = NC * NS
IDXW = 128


def _sc_gather(atab, ctab, ptab, aidx2d, cidx2d, pidx2d, dstidx, zeros, B, L):
    BPW = B // NW
    ROWS = BPW * L // IDXW
    PROWS = BPW // IDXW
    NPASS = 2
    HPW = BPW // NPASS
    ROWSP = ROWS // NPASS
    mesh = plsc.VectorSubcoreMesh(core_axis_name="c", subcore_axis_name="s")

    @functools.partial(
        pl.kernel,
        mesh=mesh,
        out_type=(
            jax.ShapeDtypeStruct((B, D), jnp.float32),
            jax.ShapeDtypeStruct((B, D), jnp.float32),
            jax.ShapeDtypeStruct((B, D), jnp.float32),
        ),
        scratch_types=[
            pltpu.VMEM((ROWS, IDXW), jnp.int32),
            pltpu.VMEM((ROWS, IDXW), jnp.int32),
            pltpu.VMEM((ROWS, IDXW), jnp.int32),
            pltpu.VMEM((PROWS, IDXW), jnp.int32),
            pltpu.VMEM((IDXW, D), jnp.float32),
            pltpu.VMEM((IDXW, D), jnp.float32),
            pltpu.VMEM((IDXW, D), jnp.float32),
            pltpu.VMEM((IDXW, D), jnp.float32),
            pltpu.VMEM_SHARED((NS * HPW, D), jnp.float32),
            pltpu.SemaphoreType.DMA,
            pltpu.SemaphoreType.DMA,
            pltpu.SemaphoreType.DMA,
            pltpu.SemaphoreType.DMA,
            pltpu.SemaphoreType.DMA,
            pltpu.SemaphoreType.DMA,
            pltpu.SemaphoreType.DMA,
            pltpu.SemaphoreType.DMA,
            pltpu.SemaphoreType.DMA,
        ],
    )
    def k(atab_h, ctab_h, ptab_h, aidx_h, cidx_h, pidx_h, dst_h, z_h,
          a_out, p_out, c_out, idx_va, idx_vc, dst_v, idx_vp,
          buf0, buf1, buf2, buf3,
          acc, g0, g1, g2, g3, s0, s1, s2, s3, msem):
        cix = lax.axis_index("c")
        six = lax.axis_index("s")
        wid = six * NC + cix
        base = wid * BPW
        bufs = (buf0, buf1, buf2, buf3)
        gsem = (g0, g1, g2, g3)
        ssem = (s0, s1, s2, s3)
        NBUF = 4
        accme = acc.at[pl.ds(six * HPW, HPW)]

        pre = [
            pltpu.async_copy(dst_h.at[six], dst_v, msem),
            pltpu.async_copy(aidx_h.at[pl.ds(wid * ROWS, ROWS)], idx_va, msem),
            pltpu.async_copy(cidx_h.at[pl.ds(wid * ROWS, ROWS)], idx_vc, msem),
            pltpu.async_copy(pidx_h.at[pl.ds(wid * PROWS, PROWS)], idx_vp, msem),
            pltpu.async_copy(z_h, accme, msem),
        ]
        for cp in pre:
            cp.wait()

        def run_pipe(n, start_gather, start_drain):
            g = [None] * n
            s = [None] * n
            issued = 0
            for j in range(n):
                while issued < n and issued < j + NBUF:
                    bi = issued % NBUF
                    if issued >= NBUF:
                        s[issued - NBUF].wait()
                    g[issued] = start_gather(issued, bufs[bi], gsem[bi])
                    issued += 1
                g[j].wait()
                s[j] = start_drain(j, bufs[j % NBUF], ssem[j % NBUF])
            for j in range(max(0, n - NBUF), n):
                s[j].wait()

        for p in range(NPASS):
            for tab_h, idx_v, out in ((atab_h, idx_va, a_out),
                                      (ctab_h, idx_vc, c_out)):
                run_pipe(
                    ROWSP,
                    lambda j, b, sm, t=tab_h, iv=idx_v, r0=p * ROWSP:
                        pltpu.async_copy(t.at[iv.at[r0 + j]], b, sm),
                    lambda j, b, sm, r0=p * ROWSP: pltpu.async_copy(
                        b, acc.at[dst_v.at[r0 + j]], sm, add=True),
                )
                pltpu.sync_copy(accme, out.at[pl.ds(base + p * HPW, HPW)])
                if not (p == NPASS - 1 and tab_h is ctab_h):
                    pltpu.sync_copy(z_h, accme)
        run_pipe(
            PROWS,
            lambda j, b, sm: pltpu.async_copy(ptab_h.at[idx_vp.at[j]], b, sm),
            lambda j, b, sm: pltpu.async_copy(
                b, p_out.at[pl.ds(base + j * IDXW, IDXW)], sm),
        )

    return k(atab, ctab, ptab, aidx2d, cidx2d, pidx2d, dstidx, zeros)


def _mlp(desc, asum, prow, csum, aidx, cidx, W1, b1, W2, b2):
    B, DD = desc.shape
    BM = 512
    L = aidx.shape[1]
    DIN, L1 = W1.shape
    L2 = W2.shape[1]

    def body(desc_ref, asum_ref, p_ref, csum_ref, aidx_ref, cidx_ref,
             W1_ref, b1_ref, W2_ref, b2_ref, out_ref):
        acnt = jnp.maximum(jnp.sum((aidx_ref[...] != 0).astype(jnp.float32),
                                   axis=1, keepdims=True), 1.0)
        ccnt = jnp.maximum(jnp.sum((cidx_ref[...] != 0).astype(jnp.float32),
                                   axis=1, keepdims=True), 1.0)
        a = asum_ref[...] / acnt
        c = csum_ref[...] / ccnt
        x = jnp.concatenate([desc_ref[...], a, p_ref[...], c], axis=1)
        h = jnp.dot(x, W1_ref[...], preferred_element_type=jnp.float32) + b1_ref[...]
        h = jnp.maximum(h, 0.0)
        out_ref[...] = jnp.dot(h, W2_ref[...],
                               preferred_element_type=jnp.float32) + b2_ref[...]

    return pl.pallas_call(
        body,
        grid=(B // BM,),
        in_specs=[
            pl.BlockSpec((BM, DD), lambda i: (i, 0)),
            pl.BlockSpec((BM, D), lambda i: (i, 0)),
            pl.BlockSpec((BM, D), lambda i: (i, 0)),
            pl.BlockSpec((BM, D), lambda i: (i, 0)),
            pl.BlockSpec((BM, L), lambda i: (i, 0)),
            pl.BlockSpec((BM, L), lambda i: (i, 0)),
            pl.BlockSpec((DIN, L1), lambda i: (0, 0)),
            pl.BlockSpec((1, L1), lambda i: (0, 0)),
            pl.BlockSpec((L1, L2), lambda i: (0, 0)),
            pl.BlockSpec((1, L2), lambda i: (0, 0)),
        ],
        out_specs=pl.BlockSpec((BM, L2), lambda i: (i, 0)),
        out_shape=jax.ShapeDtypeStruct((B, L2), jnp.float32),
    )(desc, asum, prow, csum, aidx, cidx, W1, b1.reshape(1, -1), W2,
      b2.reshape(1, -1))


def kernel(description_embedding, authors, publishers, categories,
           authors_table, publishers_table, categories_table,
           W1, b1, W2, b2):
    B, L = authors.shape
    aidx32 = authors.astype(jnp.int32)
    cidx32 = categories.astype(jnp.int32)
    aidx2d = aidx32.reshape(-1, IDXW)
    cidx2d = cidx32.reshape(-1, IDXW)
    pidx2d = publishers.astype(jnp.int32).reshape(-1, IDXW)
    BPW = B // NW
    ROWS = BPW * L // IDXW
    HPW = BPW // 2
    pat = ((jnp.arange(BPW * L, dtype=jnp.int32) // L) % HPW).reshape(ROWS, IDXW)
    dstidx = pat[None, :, :] + (jnp.arange(NS, dtype=jnp.int32) * HPW)[:, None, None]
    zeros = jnp.zeros((HPW, D), jnp.float32)
    asum, prow, csum = _sc_gather(authors_table, categories_table,
                                  publishers_table, aidx2d, cidx2d, pidx2d,
                                  dstidx, zeros, B, L)
    return _mlp(description_embedding, asum, prow, csum, aidx32, cidx32,
                W1, b1, W2, b2)
